# baseline (speedup 1.0000x reference)
import dataclasses
import functools

import jax
import jax.numpy as jnp
from jax import lax
from jax.experimental import pallas as pl
from jax.experimental.pallas import tpu as pltpu
from jax.experimental.pallas import tpu_sc as plsc

N = 10000
E = 320000
D_IN = 128
D_H = 128
D_OUT = 64
NEG = 0.2

NC = 2
NS = 16
NT = NC * NS
CH = 128
CH2 = 64
NCH = 80
EPTP = NCH * CH
EPAD = NT * EPTP
NP = 10240
RPS = NP // NS

_mesh = plsc.VectorSubcoreMesh(core_axis_name="c", subcore_axis_name="s")

_cp = pltpu.CompilerParams()
if "needs_layout_passes" in pltpu.CompilerParams.__dataclass_fields__:
    _cp = dataclasses.replace(_cp, needs_layout_passes=False)


def _leaky_exp(x):
    return jnp.exp(jnp.where(x >= 0, x, x * NEG))


def _sc_attn(el, er, src3, dst3, zeros1):
    kern = pl.kernel(
        _sc_attn_body,
        out_type=(
            jax.ShapeDtypeStruct((NT, NCH, CH), jnp.float32),
            jax.ShapeDtypeStruct((NC, NP), jnp.float32),
        ),
        mesh=_mesh,
        scratch_types=[
            pltpu.VMEM((N,), jnp.float32),
            pltpu.VMEM((N,), jnp.float32),
            pltpu.VMEM((NCH, CH), jnp.int32),
            pltpu.VMEM((NCH, CH), jnp.int32),
            pltpu.VMEM((NCH, CH), jnp.float32),
            pltpu.VMEM_SHARED((NP,), jnp.float32),
            pltpu.SemaphoreType.DMA,
        ],
        compiler_params=_cp,
    )
    return kern(el, er, src3, dst3, zeros1)


def _sc_attn_body(el_hbm, er_hbm, src_hbm, dst_hbm, z1_hbm,
                  ex_hbm, s_hbm,
                  el_v, er_v, src_v, dst_v, ex_v, s_sh, sem):
    core = lax.axis_index("c")
    sub = lax.axis_index("s")
    tile = core * NS + sub

    pltpu.sync_copy(el_hbm, el_v)
    pltpu.sync_copy(er_hbm, er_v)
    pltpu.sync_copy(src_hbm.at[tile], src_v)
    pltpu.sync_copy(dst_hbm.at[tile], dst_v)

    pltpu.sync_copy(z1_hbm.at[pl.ds(sub * RPS, RPS)], s_sh.at[pl.ds(sub * RPS, RPS)])
    plsc.subcore_barrier()

    lane = lax.iota(jnp.int32, 16)

    @pl.loop(0, NCH)
    def _(j):
        @pl.loop(0, CH, step=16)
        def _(i):
            s16 = src_v[j, pl.ds(i, 16)]
            d16 = dst_v[j, pl.ds(i, 16)]
            vals = plsc.load_gather(el_v, [s16]) + plsc.load_gather(er_v, [d16])
            gid = tile * EPTP + j * CH + i + lane
            ex_v[j, pl.ds(i, 16)] = jnp.where(gid < E, _leaky_exp(vals), 0.0)

        pltpu.sync_copy(ex_v.at[j], s_sh.at[dst_v.at[j]], add=True)

    pltpu.sync_copy(ex_v, ex_hbm.at[tile])

    plsc.subcore_barrier()
    pltpu.sync_copy(s_sh.at[pl.ds(sub * RPS, RPS)],
                    s_hbm.at[core, pl.ds(sub * RPS, RPS)])


RING = 5
TOTCHK = EPAD // CH2
K0 = 315
K1 = TOTCHK // NS - K0
ZR = (N + NS - 1) // NS // 8 * 8
ZL = N - ZR * (NS - 1)


def _sc_prop(x, srcf, dstf, exf, zeros2, d):
    kern = pl.kernel(
        functools.partial(_sc_prop_body, d=d),
        out_type=jax.ShapeDtypeStruct((NC, N, d), jnp.float32),
        mesh=_mesh,
        scratch_types=[
            pltpu.VMEM((RING, CH2), jnp.int32),
            pltpu.VMEM((RING, CH2), jnp.int32),
            pltpu.VMEM((RING, CH2), jnp.float32),
            pltpu.VMEM((RING, CH2, d), jnp.float32),
            pltpu.VMEM_SHARED((N, d), jnp.float32),
            pltpu.SemaphoreType.DMA((RING,)),
            pltpu.SemaphoreType.DMA((RING,)),
            pltpu.SemaphoreType.DMA((RING,)),
        ],
        compiler_params=_cp,
    )
    return kern(x, srcf, dstf, exf, zeros2)


def _sc_prop_body(x_hbm, src_hbm, dst_hbm, ex_hbm, z2_hbm, out_hbm,
                  src_r, dst_r, ex_r, rows_r, acc_sh, isem, gsem, ssem, *, d):
    core = lax.axis_index("c")
    sub = lax.axis_index("s")
    nchk = jnp.where(core == 0, K0, K1)
    base_chunk = jnp.where(core == 0, sub * K0, NS * K0 + sub * K1)
    ebase = base_chunk * CH2

    @pl.when(sub < NS - 1)
    def _():
        pltpu.sync_copy(z2_hbm.at[pl.ds(sub * ZR, ZR)],
                        acc_sh.at[pl.ds(sub * ZR, ZR)])

    @pl.when(sub == NS - 1)
    def _():
        pltpu.sync_copy(z2_hbm.at[pl.ds((NS - 1) * ZR, ZL)],
                        acc_sh.at[pl.ds((NS - 1) * ZR, ZL)])

    plsc.subcore_barrier()

    def i_start(c, p):
        off = ebase + c * CH2
        pltpu.async_copy(src_hbm.at[pl.ds(off, CH2)], src_r.at[p], isem.at[p])
        pltpu.async_copy(dst_hbm.at[pl.ds(off, CH2)], dst_r.at[p], isem.at[p])
        pltpu.async_copy(ex_hbm.at[pl.ds(off, CH2)], ex_r.at[p], isem.at[p])

    def i_wait(p):
        for _ in range(2):
            pltpu.make_async_copy(src_hbm.at[pl.ds(0, CH2)], src_r.at[p],
                                  isem.at[p]).wait()
        pltpu.make_async_copy(ex_hbm.at[pl.ds(0, CH2)], ex_r.at[p],
                              isem.at[p]).wait()

    def g_start(p):
        pltpu.async_copy(x_hbm.at[src_r.at[p]], rows_r.at[p], gsem.at[p])

    def g_wait(p):
        pltpu.make_async_copy(x_hbm.at[src_r.at[0]], rows_r.at[p],
                              gsem.at[p]).wait()

    def s_start(p):
        pltpu.async_copy(rows_r.at[p], acc_sh.at[dst_r.at[p]], ssem.at[p],
                         add=True)

    def s_wait(p):
        pltpu.make_async_copy(rows_r.at[0], acc_sh.at[dst_r.at[0]],
                              ssem.at[p]).wait()

    def scale(p):
        @pl.loop(0, CH2)
        def _(i):
            exb = plsc.load_gather(
                ex_r, [jnp.full((16,), p, jnp.int32), jnp.full((16,), i, jnp.int32)]
            )
            for dd in range(d // 16):
                rows_r[p, i, pl.ds(dd * 16, 16)] = (
                    rows_r[p, i, pl.ds(dd * 16, 16)] * exb)

    i_start(0, 0)
    i_start(1, 1)
    i_start(2, 2)
    i_wait(0)
    g_start(0)
    i_wait(1)
    g_start(1)

    @pl.loop(0, nchk, step=RING)
    def _(cb):
        for pp in range(RING):
            c = cb + pp

            @pl.when(c >= 2)
            def _():
                s_wait((pp - 2) % RING)

            @pl.when(c + 3 < nchk)
            def _():
                i_start(c + 3, (pp + 3) % RING)

            @pl.when(c + 2 < nchk)
            def _():
                i_wait((pp + 2) % RING)
                g_start((pp + 2) % RING)

            g_wait(pp)
            scale(pp)
            s_start(pp)

    s_wait(RING - 2)
    s_wait(RING - 1)

    plsc.subcore_barrier()

    @pl.when(sub < NS - 1)
    def _():
        pltpu.sync_copy(acc_sh.at[pl.ds(sub * ZR, ZR)],
                        out_hbm.at[core, pl.ds(sub * ZR, ZR)])

    @pl.when(sub == NS - 1)
    def _():
        pltpu.sync_copy(acc_sh.at[pl.ds((NS - 1) * ZR, ZL)],
                        out_hbm.at[core, pl.ds((NS - 1) * ZR, ZL)])


def _dot(a, b):
    return jnp.dot(a, b, preferred_element_type=jnp.float32,
                   precision=lax.Precision.HIGHEST)


def _tc_feat_body(x_ref, wg_ref, al_ref, ar_ref, feat_ref, el_ref, er_ref):
    f = _dot(x_ref[...], wg_ref[...])
    feat_ref[...] = f
    el_ref[...] = _dot(f, al_ref[...])
    er_ref[...] = _dot(f, ar_ref[...])


def _tc_feat(x, wg, al, ar):
    return pl.pallas_call(
        _tc_feat_body,
        out_shape=(
            jax.ShapeDtypeStruct((N, D_H), jnp.float32),
            jax.ShapeDtypeStruct((N, 1), jnp.float32),
            jax.ShapeDtypeStruct((N, 1), jnp.float32),
        ),
    )(x, wg, al, ar)


def _tc_h1_body(p_ref, s_ref, b_ref, o_ref):
    inv = 1.0 / (s_ref[0, :N] + s_ref[1, :N] + 1e-9)
    t = (p_ref[0, :N] + p_ref[1, :N]) * inv + b_ref[...]
    o_ref[...] = jnp.maximum(t, 0.0)


def _tc_h1(p, s, b_gat):
    return pl.pallas_call(
        _tc_h1_body,
        out_shape=jax.ShapeDtypeStruct((N, D_H), jnp.float32),
    )(p, s, b_gat)


def _tc_h2_body(q_ref, s_ref, w1_ref, b1_ref, o_ref):
    inv = 1.0 / (s_ref[0, :N] + s_ref[1, :N] + 1e-9)
    t2 = (q_ref[0] + q_ref[1]) * inv
    o_ref[...] = _dot(t2, w1_ref[...]) + b1_ref[...]


def _tc_h2(q, s, w1, b1):
    return pl.pallas_call(
        _tc_h2_body,
        out_shape=jax.ShapeDtypeStruct((N, D_H), jnp.float32),
    )(q, s, w1, b1)


def _tc_out_body(r_ref, s_ref, w2_ref, b2_ref, o_ref):
    inv = 1.0 / (s_ref[0, :N] + s_ref[1, :N] + 1e-9)
    o_ref[...] = _dot((r_ref[0] + r_ref[1]) * inv, w2_ref[...]) + b2_ref[...]


def _tc_out(r, s, w2, b2):
    return pl.pallas_call(
        _tc_out_body,
        out_shape=jax.ShapeDtypeStruct((N, D_OUT), jnp.float32),
    )(r, s, w2, b2)


def kernel(inputs, edge_index, W_gat, attn_l, attn_r, b_gat, W1, b1, W2, b2):
    pad = EPAD - E
    srcf = jnp.pad(edge_index[0], (0, pad))
    dstf = jnp.pad(edge_index[1], (0, pad))
    src3 = srcf.reshape(NT, NCH, CH)
    dst3 = dstf.reshape(NT, NCH, CH)
    zeros2 = jnp.zeros((N, D_H), jnp.float32)
    zeros1 = jnp.zeros((NP,), jnp.float32)

    feat, el2, er2 = _tc_feat(inputs, W_gat,
                              attn_l.reshape(D_H, 1), attn_r.reshape(D_H, 1))
    el = el2.reshape(N)
    er = er2.reshape(N)

    ex3, s = _sc_attn(el, er, src3, dst3, zeros1)
    s3 = s.reshape(NC, NP, 1)

    p = _sc_prop(feat, srcf, dstf, ex3.reshape(EPAD), zeros2, D_H)
    h1 = _tc_h1(p, s3, b_gat.reshape(1, D_H))

    exf = ex3.reshape(EPAD)
    q = _sc_prop(h1, srcf, dstf, exf, zeros2, D_H)
    h2 = _tc_h2(q, s3, W1, b1.reshape(1, D_H))

    r = _sc_prop(h2, srcf, dstf, exf, zeros2, D_H)
    logits = _tc_out(r, s3, W2, b2.reshape(1, D_OUT))
    return logits


# speedup vs baseline: 1.2011x; 1.2011x over previous
import dataclasses
import functools

import jax
import jax.numpy as jnp
from jax import lax
from jax.experimental import pallas as pl
from jax.experimental.pallas import tpu as pltpu
from jax.experimental.pallas import tpu_sc as plsc

N = 10000
E = 320000
D_IN = 128
D_H = 128
D_OUT = 64
NEG = 0.2

NC = 2
NS = 16
NT = NC * NS
CH = 128
CH2 = 64
NCH = 80
EPTP = NCH * CH
EPAD = NT * EPTP
NP = 10240
RPS = NP // NS

_mesh = plsc.VectorSubcoreMesh(core_axis_name="c", subcore_axis_name="s")

_cp = pltpu.CompilerParams()
if "needs_layout_passes" in pltpu.CompilerParams.__dataclass_fields__:
    _cp = dataclasses.replace(_cp, needs_layout_passes=False)


def _leaky_exp(x):
    return jnp.exp(jnp.where(x >= 0, x, x * NEG))


def _sc_attn(el, er, src3, dst3, zeros1):
    kern = pl.kernel(
        _sc_attn_body,
        out_type=(
            jax.ShapeDtypeStruct((NT, NCH, CH), jnp.float32),
            jax.ShapeDtypeStruct((NC, NP), jnp.float32),
        ),
        mesh=_mesh,
        scratch_types=[
            pltpu.VMEM((N,), jnp.float32),
            pltpu.VMEM((N,), jnp.float32),
            pltpu.VMEM((NCH, CH), jnp.int32),
            pltpu.VMEM((NCH, CH), jnp.int32),
            pltpu.VMEM((NCH, CH), jnp.float32),
            pltpu.VMEM_SHARED((NP,), jnp.float32),
            pltpu.SemaphoreType.DMA,
        ],
        compiler_params=_cp,
    )
    return kern(el, er, src3, dst3, zeros1)


def _sc_attn_body(el_hbm, er_hbm, src_hbm, dst_hbm, z1_hbm,
                  ex_hbm, s_hbm,
                  el_v, er_v, src_v, dst_v, ex_v, s_sh, sem):
    core = lax.axis_index("c")
    sub = lax.axis_index("s")
    tile = core * NS + sub

    pltpu.sync_copy(el_hbm, el_v)
    pltpu.sync_copy(er_hbm, er_v)
    pltpu.sync_copy(src_hbm.at[tile], src_v)
    pltpu.sync_copy(dst_hbm.at[tile], dst_v)

    pltpu.sync_copy(z1_hbm.at[pl.ds(sub * RPS, RPS)], s_sh.at[pl.ds(sub * RPS, RPS)])
    plsc.subcore_barrier()

    lane = lax.iota(jnp.int32, 16)

    @pl.loop(0, NCH)
    def _(j):
        @pl.loop(0, CH, step=16)
        def _(i):
            s16 = src_v[j, pl.ds(i, 16)]
            d16 = dst_v[j, pl.ds(i, 16)]
            vals = plsc.load_gather(el_v, [s16]) + plsc.load_gather(er_v, [d16])
            gid = tile * EPTP + j * CH + i + lane
            ex_v[j, pl.ds(i, 16)] = jnp.where(gid < E, _leaky_exp(vals), 0.0)

        pltpu.sync_copy(ex_v.at[j], s_sh.at[dst_v.at[j]], add=True)

    pltpu.sync_copy(ex_v, ex_hbm.at[tile])

    plsc.subcore_barrier()
    pltpu.sync_copy(s_sh.at[pl.ds(sub * RPS, RPS)],
                    s_hbm.at[core, pl.ds(sub * RPS, RPS)])


RING = 5
TOTCHK = EPAD // CH2
K0 = 305
K1 = TOTCHK // NS - K0
ZR = (N + NS - 1) // NS // 8 * 8
ZL = N - ZR * (NS - 1)


def _sc_prop(x, srcf, dstf, exf, zeros2, d):
    kern = pl.kernel(
        functools.partial(_sc_prop_body, d=d),
        out_type=jax.ShapeDtypeStruct((NC, N, d), jnp.float32),
        mesh=_mesh,
        scratch_types=[
            pltpu.VMEM((RING, CH2), jnp.int32),
            pltpu.VMEM((RING, CH2), jnp.int32),
            pltpu.VMEM((RING, CH2), jnp.float32),
            pltpu.VMEM((RING, CH2, d), jnp.float32),
            pltpu.VMEM_SHARED((N, d), jnp.float32),
            pltpu.SemaphoreType.DMA((RING,)),
            pltpu.SemaphoreType.DMA((RING,)),
            pltpu.SemaphoreType.DMA((RING,)),
        ],
        compiler_params=_cp,
    )
    return kern(x, srcf, dstf, exf, zeros2)


def _sc_prop_body(x_hbm, src_hbm, dst_hbm, ex_hbm, z2_hbm, out_hbm,
                  src_r, dst_r, ex_r, rows_r, acc_sh, isem, gsem, ssem, *, d):
    core = lax.axis_index("c")
    sub = lax.axis_index("s")
    nchk = jnp.where(core == 0, K0, K1)
    base_chunk = jnp.where(core == 0, sub * K0, NS * K0 + sub * K1)
    ebase = base_chunk * CH2

    @pl.when(sub < NS - 1)
    def _():
        pltpu.sync_copy(z2_hbm.at[pl.ds(sub * ZR, ZR)],
                        acc_sh.at[pl.ds(sub * ZR, ZR)])

    @pl.when(sub == NS - 1)
    def _():
        pltpu.sync_copy(z2_hbm.at[pl.ds((NS - 1) * ZR, ZL)],
                        acc_sh.at[pl.ds((NS - 1) * ZR, ZL)])

    plsc.subcore_barrier()

    def i_start(c, p):
        off = ebase + c * CH2
        pltpu.async_copy(src_hbm.at[pl.ds(off, CH2)], src_r.at[p], isem.at[p])
        pltpu.async_copy(dst_hbm.at[pl.ds(off, CH2)], dst_r.at[p], isem.at[p])
        pltpu.async_copy(ex_hbm.at[pl.ds(off, CH2)], ex_r.at[p], isem.at[p])

    def i_wait(p):
        for _ in range(2):
            pltpu.make_async_copy(src_hbm.at[pl.ds(0, CH2)], src_r.at[p],
                                  isem.at[p]).wait()
        pltpu.make_async_copy(ex_hbm.at[pl.ds(0, CH2)], ex_r.at[p],
                              isem.at[p]).wait()

    def g_start(p):
        pltpu.async_copy(x_hbm.at[src_r.at[p]], rows_r.at[p], gsem.at[p])

    def g_wait(p):
        pltpu.make_async_copy(x_hbm.at[src_r.at[0]], rows_r.at[p],
                              gsem.at[p]).wait()

    def s_start(p):
        pltpu.async_copy(rows_r.at[p], acc_sh.at[dst_r.at[p]], ssem.at[p],
                         add=True)

    def s_wait(p):
        pltpu.make_async_copy(rows_r.at[0], acc_sh.at[dst_r.at[0]],
                              ssem.at[p]).wait()

    def scale(p):
        @pl.loop(0, CH2)
        def _(i):
            exb = plsc.load_gather(
                ex_r, [jnp.full((16,), p, jnp.int32), jnp.full((16,), i, jnp.int32)]
            )
            for dd in range(d // 16):
                rows_r[p, i, pl.ds(dd * 16, 16)] = (
                    rows_r[p, i, pl.ds(dd * 16, 16)] * exb)

    i_start(0, 0)
    i_start(1, 1)
    i_start(2, 2)
    i_wait(0)
    g_start(0)
    i_wait(1)
    g_start(1)

    @pl.loop(0, nchk, step=RING)
    def _(cb):
        for pp in range(RING):
            c = cb + pp

            @pl.when(c >= 2)
            def _():
                s_wait((pp - 2) % RING)

            @pl.when(c + 3 < nchk)
            def _():
                i_start(c + 3, (pp + 3) % RING)

            @pl.when(c + 2 < nchk)
            def _():
                i_wait((pp + 2) % RING)
                g_start((pp + 2) % RING)

            g_wait(pp)
            scale(pp)
            s_start(pp)

    s_wait(RING - 2)
    s_wait(RING - 1)

    plsc.subcore_barrier()

    @pl.when(sub < NS - 1)
    def _():
        pltpu.sync_copy(acc_sh.at[pl.ds(sub * ZR, ZR)],
                        out_hbm.at[core, pl.ds(sub * ZR, ZR)])

    @pl.when(sub == NS - 1)
    def _():
        pltpu.sync_copy(acc_sh.at[pl.ds((NS - 1) * ZR, ZL)],
                        out_hbm.at[core, pl.ds((NS - 1) * ZR, ZL)])


def _dot(a, b):
    return jnp.dot(a, b, preferred_element_type=jnp.float32,
                   precision=lax.Precision.HIGHEST)


def _tc_feat_body(x_ref, wg_ref, al_ref, ar_ref, feat_ref, el_ref, er_ref):
    f = _dot(x_ref[...], wg_ref[...])
    feat_ref[...] = f
    el_ref[...] = _dot(f, al_ref[...])
    er_ref[...] = _dot(f, ar_ref[...])


def _tc_feat(x, wg, al, ar):
    return pl.pallas_call(
        _tc_feat_body,
        out_shape=(
            jax.ShapeDtypeStruct((N, D_H), jnp.float32),
            jax.ShapeDtypeStruct((N, 1), jnp.float32),
            jax.ShapeDtypeStruct((N, 1), jnp.float32),
        ),
    )(x, wg, al, ar)


def _tc_h1_body(p_ref, s_ref, b_ref, o_ref):
    inv = 1.0 / (s_ref[0, :N] + s_ref[1, :N] + 1e-9)
    t = (p_ref[0, :N] + p_ref[1, :N]) * inv + b_ref[...]
    o_ref[...] = jnp.maximum(t, 0.0)


def _tc_h1(p, s, b_gat):
    return pl.pallas_call(
        _tc_h1_body,
        out_shape=jax.ShapeDtypeStruct((N, D_H), jnp.float32),
    )(p, s, b_gat)


def _tc_h2_body(q_ref, s_ref, w1_ref, b1_ref, o_ref):
    inv = 1.0 / (s_ref[0, :N] + s_ref[1, :N] + 1e-9)
    t2 = (q_ref[0] + q_ref[1]) * inv
    o_ref[...] = _dot(t2, w1_ref[...]) + b1_ref[...]


def _tc_h2(q, s, w1, b1):
    return pl.pallas_call(
        _tc_h2_body,
        out_shape=jax.ShapeDtypeStruct((N, D_H), jnp.float32),
    )(q, s, w1, b1)


def _tc_out_body(r_ref, s_ref, w2_ref, b2_ref, o_ref):
    inv = 1.0 / (s_ref[0, :N] + s_ref[1, :N] + 1e-9)
    o_ref[...] = _dot((r_ref[0] + r_ref[1]) * inv, w2_ref[...]) + b2_ref[...]


def _tc_out(r, s, w2, b2):
    return pl.pallas_call(
        _tc_out_body,
        out_shape=jax.ShapeDtypeStruct((N, D_OUT), jnp.float32),
    )(r, s, w2, b2)


def kernel(inputs, edge_index, W_gat, attn_l, attn_r, b_gat, W1, b1, W2, b2):
    pad = EPAD - E
    srcf = jnp.pad(edge_index[0], (0, pad))
    dstf = jnp.pad(edge_index[1], (0, pad))
    src3 = srcf.reshape(NT, NCH, CH)
    dst3 = dstf.reshape(NT, NCH, CH)
    zeros2 = jnp.zeros((N, D_H), jnp.float32)
    zeros1 = jnp.zeros((NP,), jnp.float32)

    feat, el2, er2 = _tc_feat(inputs, W_gat,
                              attn_l.reshape(D_H, 1), attn_r.reshape(D_H, 1))
    el = el2.reshape(N)
    er = er2.reshape(N)

    ex3, s = _sc_attn(el, er, src3, dst3, zeros1)
    s3 = s.reshape(NC, NP, 1)

    p = _sc_prop(feat, srcf, dstf, ex3.reshape(EPAD), zeros2, D_H)
    h1 = _tc_h1(p, s3, b_gat.reshape(1, D_H))

    exf = ex3.reshape(EPAD)
    q = _sc_prop(h1, srcf, dstf, exf, zeros2, D_H)
    h2 = _tc_h2(q, s3, W1, b1.reshape(1, D_H))

    r = _sc_prop(h2, srcf, dstf, exf, zeros2, D_H)
    logits = _tc_out(r, s3, W2, b2.reshape(1, D_OUT))
    return logits


# speedup vs baseline: 1.2021x; 1.0009x over previous
import dataclasses
import functools

import jax
import jax.numpy as jnp
from jax import lax
from jax.experimental import pallas as pl
from jax.experimental.pallas import tpu as pltpu
from jax.experimental.pallas import tpu_sc as plsc

N = 10000
E = 320000
D_IN = 128
D_H = 128
D_OUT = 64
NEG = 0.2

NC = 2
NS = 16
NT = NC * NS
CH = 128
CH2 = 64
NCH = 80
EPTP = NCH * CH
EPAD = NT * EPTP
NP = 10240
RPS = NP // NS

_mesh = plsc.VectorSubcoreMesh(core_axis_name="c", subcore_axis_name="s")

_cp = pltpu.CompilerParams()
if "needs_layout_passes" in pltpu.CompilerParams.__dataclass_fields__:
    _cp = dataclasses.replace(_cp, needs_layout_passes=False)


def _leaky_exp(x):
    return jnp.exp(jnp.where(x >= 0, x, x * NEG))


def _sc_attn(el, er, src3, dst3, zeros1):
    kern = pl.kernel(
        _sc_attn_body,
        out_type=(
            jax.ShapeDtypeStruct((NT, NCH, CH), jnp.float32),
            jax.ShapeDtypeStruct((NC, NP), jnp.float32),
        ),
        mesh=_mesh,
        scratch_types=[
            pltpu.VMEM((N,), jnp.float32),
            pltpu.VMEM((N,), jnp.float32),
            pltpu.VMEM((NCH, CH), jnp.int32),
            pltpu.VMEM((NCH, CH), jnp.int32),
            pltpu.VMEM((NCH, CH), jnp.float32),
            pltpu.VMEM_SHARED((NP,), jnp.float32),
            pltpu.SemaphoreType.DMA,
        ],
        compiler_params=_cp,
    )
    return kern(el, er, src3, dst3, zeros1)


def _sc_attn_body(el_hbm, er_hbm, src_hbm, dst_hbm, z1_hbm,
                  ex_hbm, s_hbm,
                  el_v, er_v, src_v, dst_v, ex_v, s_sh, sem):
    core = lax.axis_index("c")
    sub = lax.axis_index("s")
    tile = core * NS + sub

    pltpu.sync_copy(el_hbm, el_v)
    pltpu.sync_copy(er_hbm, er_v)
    pltpu.sync_copy(src_hbm.at[tile], src_v)
    pltpu.sync_copy(dst_hbm.at[tile], dst_v)

    pltpu.sync_copy(z1_hbm.at[pl.ds(sub * RPS, RPS)], s_sh.at[pl.ds(sub * RPS, RPS)])
    plsc.subcore_barrier()

    lane = lax.iota(jnp.int32, 16)

    @pl.loop(0, NCH)
    def _(j):
        @pl.loop(0, CH, step=16)
        def _(i):
            s16 = src_v[j, pl.ds(i, 16)]
            d16 = dst_v[j, pl.ds(i, 16)]
            vals = plsc.load_gather(el_v, [s16]) + plsc.load_gather(er_v, [d16])
            gid = tile * EPTP + j * CH + i + lane
            ex_v[j, pl.ds(i, 16)] = jnp.where(gid < E, _leaky_exp(vals), 0.0)

        pltpu.sync_copy(ex_v.at[j], s_sh.at[dst_v.at[j]], add=True)

    pltpu.sync_copy(ex_v, ex_hbm.at[tile])

    plsc.subcore_barrier()
    pltpu.sync_copy(s_sh.at[pl.ds(sub * RPS, RPS)],
                    s_hbm.at[core, pl.ds(sub * RPS, RPS)])


RING = 5
TOTCHK = EPAD // CH2
K0 = 300
K1 = TOTCHK // NS - K0
ZR = (N + NS - 1) // NS // 8 * 8
ZL = N - ZR * (NS - 1)


def _sc_prop(x, srcf, dstf, exf, zeros2, d):
    kern = pl.kernel(
        functools.partial(_sc_prop_body, d=d),
        out_type=jax.ShapeDtypeStruct((NC, N, d), jnp.float32),
        mesh=_mesh,
        scratch_types=[
            pltpu.VMEM((RING, CH2), jnp.int32),
            pltpu.VMEM((RING, CH2), jnp.int32),
            pltpu.VMEM((RING, CH2), jnp.float32),
            pltpu.VMEM((RING, CH2, d), jnp.float32),
            pltpu.VMEM_SHARED((N, d), jnp.float32),
            pltpu.SemaphoreType.DMA((RING,)),
            pltpu.SemaphoreType.DMA((RING,)),
            pltpu.SemaphoreType.DMA((RING,)),
        ],
        compiler_params=_cp,
    )
    return kern(x, srcf, dstf, exf, zeros2)


def _sc_prop_body(x_hbm, src_hbm, dst_hbm, ex_hbm, z2_hbm, out_hbm,
                  src_r, dst_r, ex_r, rows_r, acc_sh, isem, gsem, ssem, *, d):
    core = lax.axis_index("c")
    sub = lax.axis_index("s")
    nchk = jnp.where(core == 0, K0, K1)
    base_chunk = jnp.where(core == 0, sub * K0, NS * K0 + sub * K1)
    ebase = base_chunk * CH2

    @pl.when(sub < NS - 1)
    def _():
        pltpu.sync_copy(z2_hbm.at[pl.ds(sub * ZR, ZR)],
                        acc_sh.at[pl.ds(sub * ZR, ZR)])

    @pl.when(sub == NS - 1)
    def _():
        pltpu.sync_copy(z2_hbm.at[pl.ds((NS - 1) * ZR, ZL)],
                        acc_sh.at[pl.ds((NS - 1) * ZR, ZL)])

    plsc.subcore_barrier()

    def i_start(c, p):
        off = ebase + c * CH2
        pltpu.async_copy(src_hbm.at[pl.ds(off, CH2)], src_r.at[p], isem.at[p])
        pltpu.async_copy(dst_hbm.at[pl.ds(off, CH2)], dst_r.at[p], isem.at[p])
        pltpu.async_copy(ex_hbm.at[pl.ds(off, CH2)], ex_r.at[p], isem.at[p])

    def i_wait(p):
        for _ in range(2):
            pltpu.make_async_copy(src_hbm.at[pl.ds(0, CH2)], src_r.at[p],
                                  isem.at[p]).wait()
        pltpu.make_async_copy(ex_hbm.at[pl.ds(0, CH2)], ex_r.at[p],
                              isem.at[p]).wait()

    def g_start(p):
        pltpu.async_copy(x_hbm.at[src_r.at[p]], rows_r.at[p], gsem.at[p])

    def g_wait(p):
        pltpu.make_async_copy(x_hbm.at[src_r.at[0]], rows_r.at[p],
                              gsem.at[p]).wait()

    def s_start(p):
        pltpu.async_copy(rows_r.at[p], acc_sh.at[dst_r.at[p]], ssem.at[p],
                         add=True)

    def s_wait(p):
        pltpu.make_async_copy(rows_r.at[0], acc_sh.at[dst_r.at[0]],
                              ssem.at[p]).wait()

    def scale(p):
        @pl.loop(0, CH2)
        def _(i):
            exb = plsc.load_gather(
                ex_r, [jnp.full((16,), p, jnp.int32), jnp.full((16,), i, jnp.int32)]
            )
            for dd in range(d // 16):
                rows_r[p, i, pl.ds(dd * 16, 16)] = (
                    rows_r[p, i, pl.ds(dd * 16, 16)] * exb)

    i_start(0, 0)
    i_start(1, 1)
    i_start(2, 2)
    i_wait(0)
    g_start(0)
    i_wait(1)
    g_start(1)

    @pl.loop(0, nchk, step=RING)
    def _(cb):
        for pp in range(RING):
            c = cb + pp

            @pl.when(c >= 2)
            def _():
                s_wait((pp - 2) % RING)

            @pl.when(c + 3 < nchk)
            def _():
                i_start(c + 3, (pp + 3) % RING)

            @pl.when(c + 2 < nchk)
            def _():
                i_wait((pp + 2) % RING)
                g_start((pp + 2) % RING)

            g_wait(pp)
            scale(pp)
            s_start(pp)

    s_wait(RING - 2)
    s_wait(RING - 1)

    plsc.subcore_barrier()

    @pl.when(sub < NS - 1)
    def _():
        pltpu.sync_copy(acc_sh.at[pl.ds(sub * ZR, ZR)],
                        out_hbm.at[core, pl.ds(sub * ZR, ZR)])

    @pl.when(sub == NS - 1)
    def _():
        pltpu.sync_copy(acc_sh.at[pl.ds((NS - 1) * ZR, ZL)],
                        out_hbm.at[core, pl.ds((NS - 1) * ZR, ZL)])


def _dot(a, b):
    return jnp.dot(a, b, preferred_element_type=jnp.float32,
                   precision=lax.Precision.HIGHEST)


def _tc_feat_body(x_ref, wg_ref, al_ref, ar_ref, feat_ref, el_ref, er_ref):
    f = _dot(x_ref[...], wg_ref[...])
    feat_ref[...] = f
    el_ref[...] = _dot(f, al_ref[...])
    er_ref[...] = _dot(f, ar_ref[...])


def _tc_feat(x, wg, al, ar):
    return pl.pallas_call(
        _tc_feat_body,
        out_shape=(
            jax.ShapeDtypeStruct((N, D_H), jnp.float32),
            jax.ShapeDtypeStruct((N, 1), jnp.float32),
            jax.ShapeDtypeStruct((N, 1), jnp.float32),
        ),
    )(x, wg, al, ar)


def _tc_h1_body(p_ref, s_ref, b_ref, o_ref):
    inv = 1.0 / (s_ref[0, :N] + s_ref[1, :N] + 1e-9)
    t = (p_ref[0, :N] + p_ref[1, :N]) * inv + b_ref[...]
    o_ref[...] = jnp.maximum(t, 0.0)


def _tc_h1(p, s, b_gat):
    return pl.pallas_call(
        _tc_h1_body,
        out_shape=jax.ShapeDtypeStruct((N, D_H), jnp.float32),
    )(p, s, b_gat)


def _tc_h2_body(q_ref, s_ref, w1_ref, b1_ref, o_ref):
    inv = 1.0 / (s_ref[0, :N] + s_ref[1, :N] + 1e-9)
    t2 = (q_ref[0] + q_ref[1]) * inv
    o_ref[...] = _dot(t2, w1_ref[...]) + b1_ref[...]


def _tc_h2(q, s, w1, b1):
    return pl.pallas_call(
        _tc_h2_body,
        out_shape=jax.ShapeDtypeStruct((N, D_H), jnp.float32),
    )(q, s, w1, b1)


def _tc_out_body(r_ref, s_ref, w2_ref, b2_ref, o_ref):
    inv = 1.0 / (s_ref[0, :N] + s_ref[1, :N] + 1e-9)
    o_ref[...] = _dot((r_ref[0] + r_ref[1]) * inv, w2_ref[...]) + b2_ref[...]


def _tc_out(r, s, w2, b2):
    return pl.pallas_call(
        _tc_out_body,
        out_shape=jax.ShapeDtypeStruct((N, D_OUT), jnp.float32),
    )(r, s, w2, b2)


def kernel(inputs, edge_index, W_gat, attn_l, attn_r, b_gat, W1, b1, W2, b2):
    pad = EPAD - E
    srcf = jnp.pad(edge_index[0], (0, pad))
    dstf = jnp.pad(edge_index[1], (0, pad))
    src3 = srcf.reshape(NT, NCH, CH)
    dst3 = dstf.reshape(NT, NCH, CH)
    zeros2 = jnp.zeros((N, D_H), jnp.float32)
    zeros1 = jnp.zeros((NP,), jnp.float32)

    feat, el2, er2 = _tc_feat(inputs, W_gat,
                              attn_l.reshape(D_H, 1), attn_r.reshape(D_H, 1))
    el = el2.reshape(N)
    er = er2.reshape(N)

    ex3, s = _sc_attn(el, er, src3, dst3, zeros1)
    s3 = s.reshape(NC, NP, 1)

    p = _sc_prop(feat, srcf, dstf, ex3.reshape(EPAD), zeros2, D_H)
    h1 = _tc_h1(p, s3, b_gat.reshape(1, D_H))

    exf = ex3.reshape(EPAD)
    q = _sc_prop(h1, srcf, dstf, exf, zeros2, D_H)
    h2 = _tc_h2(q, s3, W1, b1.reshape(1, D_H))

    r = _sc_prop(h2, srcf, dstf, exf, zeros2, D_H)
    logits = _tc_out(r, s3, W2, b2.reshape(1, D_OUT))
    return logits
